# baseline (speedup 1.0000x reference)
import jax
import jax.numpy as jnp
from jax.experimental import pallas as pl

_N_CHAINS = 64
_N_SITES = 8192

_ROTS = (13, 15, 26, 6, 17, 29, 16, 24)


def _threefry2x32(k0, k1, x0, x1):
    ks = (k0, k1, k0 ^ k1 ^ jnp.uint32(0x1BD11BDA))
    x0 = x0 + ks[0]
    x1 = x1 + ks[1]
    for g in range(5):
        for j in range(4):
            r = _ROTS[(g % 2) * 4 + j]
            x0 = x0 + x1
            x1 = (x1 << jnp.uint32(r)) | (x1 >> jnp.uint32(32 - r))
            x1 = x0 ^ x1
        x0 = x0 + ks[(g + 1) % 3]
        x1 = x1 + ks[(g + 2) % 3] + jnp.uint32(g + 1)
    return x0, x1


def _flip_kernel(x_ref, seeds_ref, out_ref):
    s = seeds_ref[...].astype(jnp.uint32)
    zero = jnp.zeros_like(s)
    one = jnp.ones_like(s)
    f0, f1 = _threefry2x32(zero, zero, zero, s)
    k20, k21 = _threefry2x32(f0, f1, zero, one)
    y0, y1 = _threefry2x32(k20, k21, zero, zero)
    bits = y0 ^ y1
    idx = (bits & jnp.uint32(_N_SITES - 1)).astype(jnp.int32)

    col = jax.lax.broadcasted_iota(jnp.int32, (_N_CHAINS, _N_SITES), 1)
    xv = x_ref[...]
    out_ref[...] = jnp.where(col == idx, -xv, xv)


def kernel(x, seeds):
    seeds2d = seeds.reshape(_N_CHAINS, 1)
    return pl.pallas_call(
        _flip_kernel,
        out_shape=jax.ShapeDtypeStruct((_N_CHAINS, _N_SITES), jnp.float32),
    )(x, seeds2d)


# speedup vs baseline: 1.4361x; 1.4361x over previous
import jax
import jax.numpy as jnp
from jax.experimental import pallas as pl

_N_CHAINS = 64
_N_SITES = 8192

_ROTS = (13, 15, 26, 6, 17, 29, 16, 24)


def _threefry2x32(k0, k1, x0, x1):
    ks = (k0, k1, k0 ^ k1 ^ jnp.uint32(0x1BD11BDA))
    x0 = x0 + ks[0]
    x1 = x1 + ks[1]
    for g in range(5):
        for j in range(4):
            r = _ROTS[(g % 2) * 4 + j]
            x0 = x0 + x1
            x1 = (x1 << jnp.uint32(r)) | (x1 >> jnp.uint32(32 - r))
            x1 = x0 ^ x1
        x0 = x0 + ks[(g + 1) % 3]
        x1 = x1 + ks[(g + 2) % 3] + jnp.uint32(g + 1)
    return x0, x1


def _flip_kernel(x_ref, seeds_ref, out_ref):
    s = seeds_ref[...].astype(jnp.uint32)
    zero = jnp.zeros_like(s)
    one = jnp.ones_like(s)
    f0, f1 = _threefry2x32(zero, zero, zero, s)
    k20, k21 = _threefry2x32(f0, f1, zero, one)
    y0, y1 = _threefry2x32(k20, k21, zero, zero)
    bits = y0 ^ y1
    idx = (bits & jnp.uint32(_N_SITES - 1)).astype(jnp.int32)
    idx_col = idx.reshape(_N_CHAINS, 1)

    col = jax.lax.broadcasted_iota(jnp.int32, (_N_CHAINS, _N_SITES), 1)
    xv = x_ref[...]
    out_ref[...] = jnp.where(col == idx_col, -xv, xv)


def kernel(x, seeds):
    seeds2d = seeds.reshape(1, _N_CHAINS)
    return pl.pallas_call(
        _flip_kernel,
        out_shape=jax.ShapeDtypeStruct((_N_CHAINS, _N_SITES), jnp.float32),
    )(x, seeds2d)
